# baseline (speedup 1.0000x reference)
import functools

import jax
import jax.numpy as jnp
from jax import lax
from jax.experimental import pallas as pl
from jax.experimental.pallas import tpu as pltpu
from jax.experimental.pallas import tpu_sc as plsc

N_NODES = 10000
N_EDGES = 320000
D_FEAT = 128
HIDDEN = 128
N_CLASSES = 40

ROW_BLOCK = 1000

NUM_CORES = 2
NUM_SUBCORES = 16
NUM_WORKERS = NUM_CORES * NUM_SUBCORES
CHUNK = 128
N_CHUNKS = 80
EDGES_PER_TILE = N_CHUNKS * CHUNK
N_EDGES_PAD = NUM_WORKERS * EDGES_PER_TILE
N_PAD = 10240
ROWS_PER_TILE = N_PAD // NUM_SUBCORES
DUMMY_DST = N_NODES


def _norm_mm_body(x_ref, w_ref, o_ref):
    x = x_ref[...]
    nrm = jnp.sqrt(jnp.sum(x * x, axis=1, keepdims=True))
    xn = x / jnp.maximum(nrm, 1e-12)
    o_ref[...] = jnp.dot(xn, w_ref[...], preferred_element_type=jnp.float32)


def _norm_mm(x, w):
    grid = (N_NODES // ROW_BLOCK,)
    return pl.pallas_call(
        _norm_mm_body,
        grid=grid,
        in_specs=[
            pl.BlockSpec((ROW_BLOCK, D_FEAT), lambda i: (i, 0)),
            pl.BlockSpec((D_FEAT, HIDDEN), lambda i: (0, 0)),
        ],
        out_specs=pl.BlockSpec((ROW_BLOCK, HIDDEN), lambda i: (i, 0)),
        out_shape=jax.ShapeDtypeStruct((N_NODES, HIDDEN), jnp.float32),
    )(x, w)


def _mid_body(p_ref, b_ref, w_ref, o_ref):
    h = jax.nn.relu(p_ref[0] + p_ref[1] + b_ref[...])
    nrm = jnp.sqrt(jnp.sum(h * h, axis=1, keepdims=True))
    hn = h / jnp.maximum(nrm, 1e-12)
    o_ref[...] = jnp.dot(hn, w_ref[...], preferred_element_type=jnp.float32)


def _mid(partials, b, w):
    grid = (N_NODES // ROW_BLOCK,)
    return pl.pallas_call(
        _mid_body,
        grid=grid,
        in_specs=[
            pl.BlockSpec((2, ROW_BLOCK, HIDDEN), lambda i: (0, i, 0)),
            pl.BlockSpec((1, HIDDEN), lambda i: (0, 0)),
            pl.BlockSpec((HIDDEN, HIDDEN), lambda i: (0, 0)),
        ],
        out_specs=pl.BlockSpec((ROW_BLOCK, HIDDEN), lambda i: (i, 0)),
        out_shape=jax.ShapeDtypeStruct((N_NODES, HIDDEN), jnp.float32),
    )(partials, b, w)


def _head_body(p_ref, b2_ref, wl1_ref, bl1_ref, wl2_ref, bl2_ref, o_ref):
    h = jax.nn.relu(p_ref[0] + p_ref[1] + b2_ref[...])
    h = jax.nn.relu(
        jnp.dot(h, wl1_ref[...], preferred_element_type=jnp.float32)
        + bl1_ref[...])
    z = jnp.dot(h, wl2_ref[...], preferred_element_type=jnp.float32) + bl2_ref[...]
    m = jnp.max(z, axis=-1, keepdims=True)
    e = jnp.exp(z - m)
    o_ref[...] = e / jnp.sum(e, axis=-1, keepdims=True)


def _head(partials, b2, wl1, bl1, wl2, bl2):
    grid = (N_NODES // ROW_BLOCK,)
    return pl.pallas_call(
        _head_body,
        grid=grid,
        in_specs=[
            pl.BlockSpec((2, ROW_BLOCK, HIDDEN), lambda i: (0, i, 0)),
            pl.BlockSpec((1, HIDDEN), lambda i: (0, 0)),
            pl.BlockSpec((HIDDEN, HIDDEN), lambda i: (0, 0)),
            pl.BlockSpec((1, HIDDEN), lambda i: (0, 0)),
            pl.BlockSpec((HIDDEN, N_CLASSES), lambda i: (0, 0)),
            pl.BlockSpec((1, N_CLASSES), lambda i: (0, 0)),
        ],
        out_specs=pl.BlockSpec((ROW_BLOCK, N_CLASSES), lambda i: (i, 0)),
        out_shape=jax.ShapeDtypeStruct((N_NODES, N_CLASSES), jnp.float32),
    )(partials, b2, wl1, bl1, wl2, bl2)


_SC_MESH = plsc.VectorSubcoreMesh(core_axis_name="c", subcore_axis_name="s")


@functools.partial(
    pl.kernel,
    mesh=_SC_MESH,
    out_type=jax.ShapeDtypeStruct((NUM_CORES, N_PAD, HIDDEN), jnp.float32),
    scratch_types=[
        pltpu.VMEM((N_CHUNKS, CHUNK), jnp.int32),
        pltpu.VMEM((CHUNK,), jnp.int32),
        pltpu.VMEM((CHUNK,), jnp.int32),
        pltpu.VMEM((CHUNK, HIDDEN), jnp.float32),
        pltpu.VMEM((CHUNK, HIDDEN), jnp.float32),
        pltpu.VMEM_SHARED((N_PAD, HIDDEN), jnp.float32),
        pltpu.SemaphoreType.DMA,
        pltpu.SemaphoreType.DMA,
        pltpu.SemaphoreType.DMA,
        pltpu.SemaphoreType.DMA,
        pltpu.SemaphoreType.DMA,
        pltpu.SemaphoreType.DMA,
        pltpu.SemaphoreType.DMA,
    ],
)
def _aggregate(h_hbm, src_hbm, dst_hbm, zeros_hbm, out_hbm,
               src_all, ddx0, ddx1, rows0, rows1, acc_sh,
               isem, dsem0, dsem1, gsem0, gsem1, ssem0, ssem1):
    c = lax.axis_index("c")
    s = lax.axis_index("s")
    wid = c * NUM_SUBCORES + s

    idx_g = pltpu.async_copy(src_hbm.at[wid], src_all, isem)

    pltpu.sync_copy(zeros_hbm.at[pl.ds(s * ROWS_PER_TILE, ROWS_PER_TILE)],
                    acc_sh.at[pl.ds(s * ROWS_PER_TILE, ROWS_PER_TILE)])
    idx_g.wait()
    plsc.subcore_barrier()

    def body(k, carry):
        i0 = 2 * k
        i1 = i0 + 1
        f0 = pltpu.async_copy(dst_hbm.at[wid, i0], ddx0, dsem0)
        f1 = pltpu.async_copy(dst_hbm.at[wid, i1], ddx1, dsem1)
        g0 = pltpu.async_copy(h_hbm.at[src_all.at[i0]], rows0, gsem0)
        g1 = pltpu.async_copy(h_hbm.at[src_all.at[i1]], rows1, gsem1)
        g0.wait()
        f0.wait()
        s0 = pltpu.async_copy(rows0, acc_sh.at[ddx0], ssem0, add=True)
        g1.wait()
        f1.wait()
        s1 = pltpu.async_copy(rows1, acc_sh.at[ddx1], ssem1, add=True)
        s0.wait()
        s1.wait()
        return carry

    lax.fori_loop(0, N_CHUNKS // 2, body, 0)
    plsc.subcore_barrier()

    pltpu.sync_copy(acc_sh.at[pl.ds(s * ROWS_PER_TILE, ROWS_PER_TILE)],
                    out_hbm.at[c, pl.ds(s * ROWS_PER_TILE, ROWS_PER_TILE)])


def kernel(x, edge_index, W1, b1, W2, b2, Wl1, bl1, Wl2, bl2):
    pad = N_EDGES_PAD - N_EDGES
    src = jnp.concatenate(
        [edge_index[0].astype(jnp.int32), jnp.zeros((pad,), jnp.int32)]
    ).reshape(NUM_WORKERS, N_CHUNKS, CHUNK)
    dst = jnp.concatenate(
        [edge_index[1].astype(jnp.int32), jnp.full((pad,), DUMMY_DST, jnp.int32)]
    ).reshape(NUM_WORKERS, N_CHUNKS, CHUNK)
    zeros = jnp.zeros((N_PAD, HIDDEN), jnp.float32)

    h1 = _norm_mm(x, W1)
    p1 = _aggregate(h1, src, dst, zeros)
    h2 = _mid(p1, b1.reshape(1, HIDDEN), W2)
    p2 = _aggregate(h2, src, dst, zeros)
    return _head(p2, b2.reshape(1, HIDDEN), Wl1, bl1.reshape(1, HIDDEN),
                 Wl2, bl2.reshape(1, N_CLASSES))


# speedup vs baseline: 1.1784x; 1.1784x over previous
import functools

import jax
import jax.numpy as jnp
from jax import lax
from jax.experimental import pallas as pl
from jax.experimental.pallas import tpu as pltpu
from jax.experimental.pallas import tpu_sc as plsc

N_NODES = 10000
N_EDGES = 320000
D_FEAT = 128
HIDDEN = 128
N_CLASSES = 40

ROW_BLOCK = 1000

NUM_CORES = 2
NUM_SUBCORES = 16
NUM_WORKERS = NUM_CORES * NUM_SUBCORES
CHUNK = 128
N_CHUNKS = 80
EDGES_PER_TILE = N_CHUNKS * CHUNK
N_EDGES_PAD = NUM_WORKERS * EDGES_PER_TILE
N_PAD = 10240
ROWS_PER_TILE = N_PAD // NUM_SUBCORES
DUMMY_DST = N_NODES


def _norm_mm_body(x_ref, w_ref, o_ref):
    x = x_ref[...]
    nrm = jnp.sqrt(jnp.sum(x * x, axis=1, keepdims=True))
    xn = x / jnp.maximum(nrm, 1e-12)
    o_ref[...] = jnp.dot(xn, w_ref[...], preferred_element_type=jnp.float32)


def _norm_mm(x, w):
    grid = (N_NODES // ROW_BLOCK,)
    return pl.pallas_call(
        _norm_mm_body,
        grid=grid,
        in_specs=[
            pl.BlockSpec((ROW_BLOCK, D_FEAT), lambda i: (i, 0)),
            pl.BlockSpec((D_FEAT, HIDDEN), lambda i: (0, 0)),
        ],
        out_specs=pl.BlockSpec((ROW_BLOCK, HIDDEN), lambda i: (i, 0)),
        out_shape=jax.ShapeDtypeStruct((N_NODES, HIDDEN), jnp.float32),
    )(x, w)


def _mid_body(p_ref, b_ref, w_ref, o_ref):
    h = jax.nn.relu(p_ref[0] + p_ref[1] + b_ref[...])
    nrm = jnp.sqrt(jnp.sum(h * h, axis=1, keepdims=True))
    hn = h / jnp.maximum(nrm, 1e-12)
    o_ref[...] = jnp.dot(hn, w_ref[...], preferred_element_type=jnp.float32)


def _mid(partials, b, w):
    grid = (N_NODES // ROW_BLOCK,)
    return pl.pallas_call(
        _mid_body,
        grid=grid,
        in_specs=[
            pl.BlockSpec((2, ROW_BLOCK, HIDDEN), lambda i: (0, i, 0)),
            pl.BlockSpec((1, HIDDEN), lambda i: (0, 0)),
            pl.BlockSpec((HIDDEN, HIDDEN), lambda i: (0, 0)),
        ],
        out_specs=pl.BlockSpec((ROW_BLOCK, HIDDEN), lambda i: (i, 0)),
        out_shape=jax.ShapeDtypeStruct((N_NODES, HIDDEN), jnp.float32),
    )(partials, b, w)


def _head_body(p_ref, b2_ref, wl1_ref, bl1_ref, wl2_ref, bl2_ref, o_ref):
    h = jax.nn.relu(p_ref[0] + p_ref[1] + b2_ref[...])
    h = jax.nn.relu(
        jnp.dot(h, wl1_ref[...], preferred_element_type=jnp.float32)
        + bl1_ref[...])
    z = jnp.dot(h, wl2_ref[...], preferred_element_type=jnp.float32) + bl2_ref[...]
    m = jnp.max(z, axis=-1, keepdims=True)
    e = jnp.exp(z - m)
    o_ref[...] = e / jnp.sum(e, axis=-1, keepdims=True)


def _head(partials, b2, wl1, bl1, wl2, bl2):
    grid = (N_NODES // ROW_BLOCK,)
    return pl.pallas_call(
        _head_body,
        grid=grid,
        in_specs=[
            pl.BlockSpec((2, ROW_BLOCK, HIDDEN), lambda i: (0, i, 0)),
            pl.BlockSpec((1, HIDDEN), lambda i: (0, 0)),
            pl.BlockSpec((HIDDEN, HIDDEN), lambda i: (0, 0)),
            pl.BlockSpec((1, HIDDEN), lambda i: (0, 0)),
            pl.BlockSpec((HIDDEN, N_CLASSES), lambda i: (0, 0)),
            pl.BlockSpec((1, N_CLASSES), lambda i: (0, 0)),
        ],
        out_specs=pl.BlockSpec((ROW_BLOCK, N_CLASSES), lambda i: (i, 0)),
        out_shape=jax.ShapeDtypeStruct((N_NODES, N_CLASSES), jnp.float32),
    )(partials, b2, wl1, bl1, wl2, bl2)


_SC_MESH = plsc.VectorSubcoreMesh(core_axis_name="c", subcore_axis_name="s")


@functools.partial(
    pl.kernel,
    mesh=_SC_MESH,
    out_type=jax.ShapeDtypeStruct((NUM_CORES, N_PAD, HIDDEN), jnp.float32),
    scratch_types=[
        pltpu.VMEM((N_CHUNKS, CHUNK), jnp.int32),
        pltpu.VMEM((CHUNK,), jnp.int32),
        pltpu.VMEM((CHUNK,), jnp.int32),
        pltpu.VMEM((CHUNK, HIDDEN), jnp.float32),
        pltpu.VMEM((CHUNK, HIDDEN), jnp.float32),
        pltpu.VMEM_SHARED((N_PAD, HIDDEN), jnp.float32),
        pltpu.SemaphoreType.DMA,
        pltpu.SemaphoreType.DMA,
        pltpu.SemaphoreType.DMA,
        pltpu.SemaphoreType.DMA,
        pltpu.SemaphoreType.DMA,
        pltpu.SemaphoreType.DMA,
        pltpu.SemaphoreType.DMA,
    ],
)
def _aggregate(h_hbm, src_hbm, dst_hbm, zeros_hbm, out_hbm,
               src_all, ddx0, ddx1, rows0, rows1, acc_sh,
               isem, dsem0, dsem1, gsem0, gsem1, ssem0, ssem1):
    c = lax.axis_index("c")
    s = lax.axis_index("s")
    wid = c * NUM_SUBCORES + s

    idx_g = pltpu.async_copy(src_hbm.at[wid], src_all, isem)

    pltpu.sync_copy(zeros_hbm.at[pl.ds(s * ROWS_PER_TILE, ROWS_PER_TILE)],
                    acc_sh.at[pl.ds(s * ROWS_PER_TILE, ROWS_PER_TILE)])
    idx_g.wait()
    plsc.subcore_barrier()

    def body(k, carry):
        i0 = 2 * k
        i1 = i0 + 1
        f0 = pltpu.async_copy(dst_hbm.at[wid, i0], ddx0, dsem0)
        f1 = pltpu.async_copy(dst_hbm.at[wid, i1], ddx1, dsem1)
        g0 = pltpu.async_copy(h_hbm.at[src_all.at[i0]], rows0, gsem0)
        g1 = pltpu.async_copy(h_hbm.at[src_all.at[i1]], rows1, gsem1)
        g0.wait()
        f0.wait()
        s0 = pltpu.async_copy(rows0, acc_sh.at[ddx0], ssem0, add=True)
        g1.wait()
        f1.wait()
        s1 = pltpu.async_copy(rows1, acc_sh.at[ddx1], ssem1, add=True)
        s0.wait()
        s1.wait()
        return carry

    lax.fori_loop(0, N_CHUNKS // 2, body, 0)
    plsc.subcore_barrier()

    pltpu.sync_copy(acc_sh.at[pl.ds(s * ROWS_PER_TILE, ROWS_PER_TILE)],
                    out_hbm.at[c, pl.ds(s * ROWS_PER_TILE, ROWS_PER_TILE)])


def kernel(x, edge_index, W1, b1, W2, b2, Wl1, bl1, Wl2, bl2):
    real_per_tile = N_EDGES // NUM_WORKERS
    pad_per_tile = EDGES_PER_TILE - real_per_tile
    src_r = edge_index[0].astype(jnp.int32).reshape(NUM_WORKERS, real_per_tile)
    dst_r = edge_index[1].astype(jnp.int32).reshape(NUM_WORKERS, real_per_tile)
    dummy_dst = jnp.broadcast_to(
        DUMMY_DST + jnp.arange(pad_per_tile, dtype=jnp.int32),
        (NUM_WORKERS, pad_per_tile))
    src = jnp.concatenate(
        [src_r, jnp.zeros((NUM_WORKERS, pad_per_tile), jnp.int32)], axis=1
    ).reshape(NUM_WORKERS, N_CHUNKS, CHUNK)
    dst = jnp.concatenate([dst_r, dummy_dst], axis=1
    ).reshape(NUM_WORKERS, N_CHUNKS, CHUNK)
    zeros = jnp.zeros((N_PAD, HIDDEN), jnp.float32)

    h1 = _norm_mm(x, W1)
    p1 = _aggregate(h1, src, dst, zeros)
    h2 = _mid(p1, b1.reshape(1, HIDDEN), W2)
    p2 = _aggregate(h2, src, dst, zeros)
    return _head(p2, b2.reshape(1, HIDDEN), Wl1, bl1.reshape(1, HIDDEN),
                 Wl2, bl2.reshape(1, N_CLASSES))
